# baseline (speedup 1.0000x reference)
import jax
import jax.numpy as jnp
from jax.experimental import pallas as pl

_NUM_SLOTS = 9


def _fill_kernel(in_ref, out_ref):
    rb = in_ref.shape[0]
    F = out_ref.shape[1] // _NUM_SLOTS
    flat = in_ref[...].reshape(rb, F)
    out_ref[:, :F] = flat
    out_ref[:, F:(_NUM_SLOTS - 1) * F] = jnp.zeros_like(
        out_ref[:, F:(_NUM_SLOTS - 1) * F]
    )
    out_ref[:, (_NUM_SLOTS - 1) * F:] = flat


def kernel(features):
    B, D, W = features.shape
    F = D * W
    rb = 8
    return pl.pallas_call(
        _fill_kernel,
        grid=(B // rb,),
        in_specs=[pl.BlockSpec((rb, D, W), lambda i: (i, 0, 0))],
        out_specs=pl.BlockSpec((rb, _NUM_SLOTS * F), lambda i: (i, 0)),
        out_shape=jax.ShapeDtypeStruct((B, _NUM_SLOTS * F), features.dtype),
    )(features)


# speedup vs baseline: 1.0028x; 1.0028x over previous
import jax
import jax.numpy as jnp
from jax.experimental import pallas as pl

_NUM_SLOTS = 9


def _fill_kernel(in_ref, out_ref):
    rb = in_ref.shape[0]
    F = out_ref.shape[1] // _NUM_SLOTS
    flat = in_ref[...].reshape(rb, F)
    out_ref[:, :F] = flat
    out_ref[:, F:(_NUM_SLOTS - 1) * F] = jnp.zeros_like(
        out_ref[:, F:(_NUM_SLOTS - 1) * F]
    )
    out_ref[:, (_NUM_SLOTS - 1) * F:] = flat


def kernel(features):
    B, D, W = features.shape
    F = D * W
    rb = 16
    return pl.pallas_call(
        _fill_kernel,
        grid=(B // rb,),
        in_specs=[pl.BlockSpec((rb, D, W), lambda i: (i, 0, 0))],
        out_specs=pl.BlockSpec((rb, _NUM_SLOTS * F), lambda i: (i, 0)),
        out_shape=jax.ShapeDtypeStruct((B, _NUM_SLOTS * F), features.dtype),
    )(features)
